# baseline (speedup 1.0000x reference)
import jax
import jax.numpy as jnp
from jax.experimental import pallas as pl

_VOCAB = 1000
_ROWS_PER_BLK = 64


def _onehot_block(x_ref, o_ref):
    o_ref[...] = jnp.zeros(o_ref.shape, jnp.float32)


def kernel(x, W):
    B, H = x.shape
    x3 = x.reshape(B, H, 1).astype(jnp.int32)
    G = B // _ROWS_PER_BLK
    out = pl.pallas_call(
        _onehot_block,
        grid=(G,),
        in_specs=[pl.BlockSpec((_ROWS_PER_BLK, H, 1), lambda i: (i, 0, 0))],
        out_specs=pl.BlockSpec((_ROWS_PER_BLK, H, _VOCAB), lambda i: (i, 0, 0)),
        out_shape=jax.ShapeDtypeStruct((B, H, _VOCAB), jnp.float32),
    )(x3)
    return out


# speedup vs baseline: 1.0914x; 1.0914x over previous
import jax
import jax.numpy as jnp
from jax.experimental import pallas as pl
from jax.experimental.pallas import tpu as pltpu

_VOCAB = 1000
_RB = 32
_K = 8


def _onehot_body(x_ref, o_hbm, *rest):
    bufs = rest[:_K]
    sems = rest[_K:]
    s = pl.program_id(0)
    n_steps = pl.num_programs(0)
    H = x_ref.shape[1]

    for j in range(_K):
        idx = x_ref[pl.ds(j * _RB, _RB), :]
        idx3 = idx[:, :, None]
        iota = jax.lax.broadcasted_iota(jnp.int32, (_RB, H, _VOCAB), 2)

        @pl.when(s > 0)
        def _wait_prev(j=j):
            pltpu.make_async_copy(
                bufs[j], o_hbm.at[pl.ds(0, _RB)], sems[j]
            ).wait()

        bufs[j][...] = (idx3 == iota).astype(jnp.float32)
        row0 = (s * _K + j) * _RB
        pltpu.make_async_copy(
            bufs[j], o_hbm.at[pl.ds(row0, _RB)], sems[j]
        ).start()

    @pl.when(s == n_steps - 1)
    def _drain():
        for j in range(_K):
            pltpu.make_async_copy(
                bufs[j], o_hbm.at[pl.ds(0, _RB)], sems[j]
            ).wait()


def kernel(x, W):
    B, H = x.shape
    xi = x.astype(jnp.int32)
    steps = B // (_K * _RB)
    out = pl.pallas_call(
        _onehot_body,
        grid=(steps,),
        in_specs=[pl.BlockSpec((_K * _RB, H), lambda i: (i, 0))],
        out_specs=pl.BlockSpec(memory_space=pl.ANY),
        out_shape=jax.ShapeDtypeStruct((B, H, _VOCAB), jnp.float32),
        scratch_shapes=(
            [pltpu.VMEM((_RB, H, _VOCAB), jnp.float32) for _ in range(_K)]
            + [pltpu.SemaphoreType.DMA for _ in range(_K)]
        ),
    )(xi)
    return out
